# baseline (speedup 1.0000x reference)
import functools

import jax
import jax.numpy as jnp
from jax.experimental import pallas as pl
from jax.experimental.pallas import tpu as pltpu

N_NODES = 100000
IN_C = 128
OUT_C = 128
DENSE_OUT = 256
EMB_DIM = 64

SLICE_R = 4000
NSLICE = N_NODES // SLICE_R
NBUF = 25


def _gnn_kernel(x_hbm, w0_ref, w1_ref, w2_ref, cb_ref, dw_ref, db_ref,
                ew_ref, eb_ref, y_ref, bufs, sems):
    def start(k):
        return pltpu.make_async_copy(
            x_hbm.at[pl.ds(k * SLICE_R, SLICE_R), :], bufs.at[k % NBUF],
            sems.at[k % NBUF])

    for k in range(NBUF):
        start(k).start()

    acc = jnp.zeros((1, IN_C), jnp.float32)
    x0 = None
    for k in range(NSLICE):
        start(k).wait()
        if k == 0:
            x0 = bufs[0, 0:1, :]
        acc = acc + jnp.sum(bufs[k % NBUF], axis=0, keepdims=True)
        if k + NBUF < NSLICE:
            start(k + NBUF).start()

    inv_n = 1.0 / N_NODES
    colmean = acc * inv_n
    w_mean = w0_ref[...] - w2_ref[...]
    w_corr = 18.0 * w2_ref[...] - 3.0 * w1_ref[...]
    dn = (((1,), (1,)), ((), ()))
    pooled = (
        jax.lax.dot_general(colmean, w_mean, dn,
                            preferred_element_type=jnp.float32)
        + inv_n * jax.lax.dot_general(x0, w_corr, dn,
                                      preferred_element_type=jnp.float32)
        + cb_ref[...]
    )
    h = jax.lax.dot_general(pooled, dw_ref[...], dn,
                            preferred_element_type=jnp.float32) + db_ref[...]
    y = jax.lax.dot_general(h, ew_ref[...], dn,
                            preferred_element_type=jnp.float32) + eb_ref[...]
    y_ref[...] = y


@jax.jit
def kernel(x, cheb_W0, cheb_W1, cheb_W2, cheb_b, dense_W, dense_b, emb_W,
           emb_b):
    cb = cheb_b.reshape(1, OUT_C)
    db = dense_b.reshape(1, DENSE_OUT)
    eb = emb_b.reshape(1, EMB_DIM)

    full = lambda shape: pl.BlockSpec(shape, lambda i: (0,) * len(shape))
    return pl.pallas_call(
        _gnn_kernel,
        grid=(1,),
        in_specs=[
            pl.BlockSpec(memory_space=pl.ANY),
            full((OUT_C, IN_C)),
            full((OUT_C, IN_C)),
            full((OUT_C, IN_C)),
            full((1, OUT_C)),
            full((DENSE_OUT, OUT_C)),
            full((1, DENSE_OUT)),
            full((EMB_DIM, DENSE_OUT)),
            full((1, EMB_DIM)),
        ],
        out_specs=pl.BlockSpec((1, EMB_DIM), lambda i: (0, 0)),
        out_shape=jax.ShapeDtypeStruct((1, EMB_DIM), jnp.float32),
        scratch_shapes=[
            pltpu.VMEM((NBUF, SLICE_R, IN_C), jnp.float32),
            pltpu.SemaphoreType.DMA((NBUF,)),
        ],
    )(x, cheb_W0, cheb_W1, cheb_W2, cb, dense_W, db, emb_W, eb)


# speedup vs baseline: 1.0418x; 1.0418x over previous
import functools

import jax
import jax.numpy as jnp
from jax.experimental import pallas as pl
from jax.experimental.pallas import tpu as pltpu

N_NODES = 100000
IN_C = 128
OUT_C = 128
DENSE_OUT = 256
EMB_DIM = 64

SLICE_R = 4000
NSLICE = N_NODES // SLICE_R
NBUF = 4


def _gnn_kernel(x_hbm, w0_ref, w1_ref, w2_ref, cb_ref, dw_ref, db_ref,
                ew_ref, eb_ref, y_ref, bufs, sems):
    def start(k):
        return pltpu.make_async_copy(
            x_hbm.at[pl.ds(k * SLICE_R, SLICE_R), :], bufs.at[k % NBUF],
            sems.at[k % NBUF])

    for k in range(NBUF):
        start(k).start()

    acc = jnp.zeros((1, IN_C), jnp.float32)
    x0 = None
    for k in range(NSLICE):
        start(k).wait()
        if k == 0:
            x0 = bufs[0, 0:1, :]
        acc = acc + jnp.sum(bufs[k % NBUF], axis=0, keepdims=True)
        if k + NBUF < NSLICE:
            start(k + NBUF).start()

    inv_n = 1.0 / N_NODES
    colmean = acc * inv_n
    w_mean = w0_ref[...] - w2_ref[...]
    w_corr = 18.0 * w2_ref[...] - 3.0 * w1_ref[...]
    dn = (((1,), (1,)), ((), ()))
    pooled = (
        jax.lax.dot_general(colmean, w_mean, dn,
                            preferred_element_type=jnp.float32)
        + inv_n * jax.lax.dot_general(x0, w_corr, dn,
                                      preferred_element_type=jnp.float32)
        + cb_ref[...]
    )
    h = jax.lax.dot_general(pooled, dw_ref[...], dn,
                            preferred_element_type=jnp.float32) + db_ref[...]
    y = jax.lax.dot_general(h, ew_ref[...], dn,
                            preferred_element_type=jnp.float32) + eb_ref[...]
    y_ref[...] = y


@jax.jit
def kernel(x, cheb_W0, cheb_W1, cheb_W2, cheb_b, dense_W, dense_b, emb_W,
           emb_b):
    cb = cheb_b.reshape(1, OUT_C)
    db = dense_b.reshape(1, DENSE_OUT)
    eb = emb_b.reshape(1, EMB_DIM)

    full = lambda shape: pl.BlockSpec(shape, lambda i: (0,) * len(shape))
    return pl.pallas_call(
        _gnn_kernel,
        grid=(1,),
        in_specs=[
            pl.BlockSpec(memory_space=pl.ANY),
            full((OUT_C, IN_C)),
            full((OUT_C, IN_C)),
            full((OUT_C, IN_C)),
            full((1, OUT_C)),
            full((DENSE_OUT, OUT_C)),
            full((1, DENSE_OUT)),
            full((EMB_DIM, DENSE_OUT)),
            full((1, EMB_DIM)),
        ],
        out_specs=pl.BlockSpec((1, EMB_DIM), lambda i: (0, 0)),
        out_shape=jax.ShapeDtypeStruct((1, EMB_DIM), jnp.float32),
        scratch_shapes=[
            pltpu.VMEM((NBUF, SLICE_R, IN_C), jnp.float32),
            pltpu.SemaphoreType.DMA((NBUF,)),
        ],
    )(x, cheb_W0, cheb_W1, cheb_W2, cb, dense_W, db, emb_W, eb)


# speedup vs baseline: 1.0446x; 1.0027x over previous
import jax
import jax.numpy as jnp
from jax.experimental import pallas as pl
from jax.experimental.pallas import tpu as pltpu

N_NODES = 100000
IN_C = 128
OUT_C = 128
DENSE_OUT = 256
EMB_DIM = 64

SLICE_R = 4000
NSLICE = N_NODES // SLICE_R
NBUF = 4


def _gnn_kernel(x_hbm, w0_ref, w1_ref, w2_ref, cb_ref, dw_ref, db_ref,
                ew_ref, eb_ref, y_ref, bufs, sems):
    def start(k):
        return pltpu.make_async_copy(
            x_hbm.at[pl.ds(k * SLICE_R, SLICE_R), :], bufs.at[k % NBUF],
            sems.at[k % NBUF])

    for k in range(NBUF):
        start(k).start()

    acc = jnp.zeros((1, IN_C), jnp.float32)
    x0 = None
    for k in range(NSLICE):
        start(k).wait()
        if k == 0:
            x0 = bufs[0, 0:1, :]
        acc = acc + jnp.sum(bufs[k % NBUF], axis=0, keepdims=True)
        if k + NBUF < NSLICE:
            start(k + NBUF).start()

    inv_n = 1.0 / N_NODES
    colmean = acc * inv_n
    w_mean = w0_ref[...] - w2_ref[...]
    w_corr = 18.0 * w2_ref[...] - 3.0 * w1_ref[...]
    dn = (((1,), (1,)), ((), ()))
    pooled = (
        jax.lax.dot_general(colmean, w_mean, dn,
                            preferred_element_type=jnp.float32)
        + inv_n * jax.lax.dot_general(x0, w_corr, dn,
                                      preferred_element_type=jnp.float32)
        + cb_ref[...]
    )
    h = jax.lax.dot_general(pooled, dw_ref[...], dn,
                            preferred_element_type=jnp.float32) + db_ref[...]
    y = jax.lax.dot_general(h, ew_ref[...], dn,
                            preferred_element_type=jnp.float32) + eb_ref[...]
    y_ref[...] = y


@jax.jit
def kernel(x, cheb_W0, cheb_W1, cheb_W2, cheb_b, dense_W, dense_b, emb_W,
           emb_b):
    cb = cheb_b.reshape(1, OUT_C)
    db = dense_b.reshape(1, DENSE_OUT)
    eb = emb_b.reshape(1, EMB_DIM)

    full = lambda shape: pl.BlockSpec(shape, lambda i: (0,) * len(shape))
    return pl.pallas_call(
        _gnn_kernel,
        grid=(1,),
        in_specs=[
            pl.BlockSpec(memory_space=pl.ANY),
            full((OUT_C, IN_C)),
            full((OUT_C, IN_C)),
            full((OUT_C, IN_C)),
            full((1, OUT_C)),
            full((DENSE_OUT, OUT_C)),
            full((1, DENSE_OUT)),
            full((EMB_DIM, DENSE_OUT)),
            full((1, EMB_DIM)),
        ],
        out_specs=pl.BlockSpec((1, EMB_DIM), lambda i: (0, 0)),
        out_shape=jax.ShapeDtypeStruct((1, EMB_DIM), jnp.float32),
        scratch_shapes=[
            pltpu.VMEM((NBUF, SLICE_R, IN_C), jnp.float32),
            pltpu.SemaphoreType.DMA((NBUF,)),
        ],
    )(x, cheb_W0, cheb_W1, cheb_W2, cb, dense_W, db, emb_W, eb)
